# baseline (speedup 1.0000x reference)
import functools

import jax
import jax.numpy as jnp
from jax import lax
from jax.experimental import pallas as pl
from jax.experimental.pallas import tpu as pltpu
from jax.experimental.pallas import tpu_sc as plsc

N_USER = 5000
N_ITEM = 5000
N = N_USER + N_ITEM
E = 160000
D = 256
R = 4
NB = 2
HID = 256
NTILE = 16
NSC = 2
NPAD = 10112
OWN = NPAD // NTILE
DUMP = R * NPAD
PLANE = DUMP + NTILE
EHALF = E // NSC
EB = 2000
BATCH = 128
CB = OWN
ZR = 64


def _sc_body(x_hbm, src_hbm, dst_hbm, typ_hbm, zeros_hbm, s_out, cnt_out,
             es, ed, et, gbuf, rbuf, kbuf, g2buf, r2buf, rows_v, sacc, zbuf,
             cntv):
    c = lax.axis_index("c")
    s = lax.axis_index("s")
    lo = s * OWN
    plane = c * PLANE

    pltpu.sync_copy(zeros_hbm, zbuf)
    for t in range(R):
        base = plane + t * NPAD + lo
        off = 0
        for nz in (ZR, ZR, ZR, ZR, ZR, ZR, ZR, ZR, ZR, OWN - 9 * ZR):
            pltpu.sync_copy(zbuf.at[pl.ds(0, nz)],
                            s_out.at[pl.ds(base + off, nz)])
            off += nz

    def zc(i, carry):
        cntv[pl.ds(i * 16, 16)] = jnp.zeros((16,), jnp.float32)
        return carry

    lax.fori_loop(0, (R * CB) // 16, zc, jnp.int32(0))

    ebase = c * EHALF
    zero16 = jnp.zeros((16,), jnp.int32)
    dump16 = jnp.full((16,), DUMP, jnp.int32) + plane + s

    def edge_batch(eb, carry):
        eoff = ebase + eb * EB
        pltpu.sync_copy(src_hbm.at[pl.ds(eoff, EB)], es)
        pltpu.sync_copy(dst_hbm.at[pl.ds(eoff, EB)], ed)
        pltpu.sync_copy(typ_hbm.at[pl.ds(eoff, EB)], et)

        def filt(i, m):
            d16 = ed[pl.ds(i * 16, 16)]
            msk = (d16 >= lo) & (d16 < lo + OWN)
            s16 = es[pl.ds(i * 16, 16)]
            t16 = et[pl.ds(i * 16, 16)]
            rowid = plane + t16 * NPAD + d16
            inc = msk.astype(jnp.int32)
            cs = plsc.cumsum(inc)
            pos = m + cs - inc
            plsc.store_scatter(gbuf, [pos], s16, mask=msk)
            plsc.store_scatter(rbuf, [pos], rowid, mask=msk)
            lidx = t16 * CB + (d16 - lo)
            rc, lastm = plsc.scan_count(lidx, mask=msk)
            okm = lastm & msk
            curall = plsc.load_gather(cntv, [lidx], mask=msk)
            grank = curall.astype(jnp.int32) + rc - 1
            plsc.store_scatter(kbuf, [pos], grank, mask=msk)
            cur = plsc.load_gather(cntv, [lidx], mask=okm)
            plsc.store_scatter(cntv, [lidx], cur + rc.astype(jnp.float32),
                               mask=okm)
            return m + jnp.sum(inc)

        m = lax.fori_loop(0, EB // 16, filt, jnp.int32(0))

        iota16 = lax.iota(jnp.int32, 16)
        neg16 = jnp.full((16,), -1, jnp.int32)
        for k in range(BATCH // 16):
            kbuf[pl.ds(m + k * 16, 16)] = neg16

        nv = (m + 15) // 16

        def round_body(state):
            k, rem = state

            def rfilt(i, m2):
                kb16 = kbuf[pl.ds(i * 16, 16)]
                msk2 = kb16 == k
                g16 = gbuf[pl.ds(i * 16, 16)]
                r16 = rbuf[pl.ds(i * 16, 16)]
                inc2 = msk2.astype(jnp.int32)
                cs2 = plsc.cumsum(inc2)
                pos2 = m2 + cs2 - inc2
                plsc.store_scatter(g2buf, [pos2], g16, mask=msk2)
                plsc.store_scatter(r2buf, [pos2 // BATCH, pos2 % BATCH],
                                   r16, mask=msk2)
                return m2 + jnp.sum(inc2)

            m2 = lax.fori_loop(0, nv, rfilt, jnp.int32(0))

            for kk in range(BATCH // 16):
                g2buf[pl.ds(m2 + kk * 16, 16)] = zero16
                p2 = m2 + kk * 16 + iota16
                plsc.store_scatter(r2buf, [p2 // BATCH, p2 % BATCH], dump16)

            def gs(b, carry2):
                off = b * BATCH
                pltpu.sync_copy(s_out.at[r2buf.at[b]], sacc)
                pltpu.sync_copy(x_hbm.at[g2buf.at[pl.ds(off, BATCH)]], rows_v)

                def addrow(i, c3):
                    for kk in range(D // 16):
                        sl = pl.ds(kk * 16, 16)
                        sacc[i, sl] = sacc[i, sl] + rows_v[i, sl]
                    return c3

                lax.fori_loop(0, BATCH, addrow, jnp.int32(0))
                pltpu.sync_copy(sacc, s_out.at[r2buf.at[b]])
                return carry2

            nb = (m2 + BATCH - 1) // BATCH
            lax.fori_loop(0, nb, gs, jnp.int32(0))
            return (k + 1, rem - m2)

        lax.while_loop(lambda st: st[1] > 0, round_body, (jnp.int32(0), m))
        return carry

    lax.fori_loop(0, EHALF // EB, edge_batch, jnp.int32(0))

    pltpu.sync_copy(cntv, cnt_out.at[c * NTILE + s])


def _sc_segment_sums(x, src, dst, typ, zeros_z):
    f = pl.kernel(
        _sc_body,
        out_type=(
            jax.ShapeDtypeStruct((NSC * PLANE, D), jnp.float32),
            jax.ShapeDtypeStruct((NSC * NTILE, R * CB), jnp.float32),
        ),
        mesh=plsc.VectorSubcoreMesh(core_axis_name="c", subcore_axis_name="s"),
        scratch_types=[
            pltpu.VMEM((EB,), jnp.int32),
            pltpu.VMEM((EB,), jnp.int32),
            pltpu.VMEM((EB,), jnp.int32),
            pltpu.VMEM((EB + BATCH,), jnp.int32),
            pltpu.VMEM((EB + BATCH,), jnp.int32),
            pltpu.VMEM((EB + BATCH,), jnp.int32),
            pltpu.VMEM((EB + BATCH,), jnp.int32),
            pltpu.VMEM(((EB + BATCH) // BATCH, BATCH), jnp.int32),
            pltpu.VMEM((BATCH, D), jnp.float32),
            pltpu.VMEM((BATCH, D), jnp.float32),
            pltpu.VMEM((ZR, D), jnp.float32),
            pltpu.VMEM((R * CB,), jnp.float32),
        ],
        compiler_params=pltpu.CompilerParams(needs_layout_passes=False),
    )
    return f(x, src, dst, typ, zeros_z)


def _proj_body(x_ref, w_ref, b_ref, o_ref):
    h = jnp.dot(x_ref[...], w_ref[...], preferred_element_type=jnp.float32)
    o_ref[...] = jnp.maximum(h + b_ref[...], 0.0)


def _project(x, W, b):
    n, bm = x.shape[0], 1000
    return pl.pallas_call(
        _proj_body,
        grid=(n // bm,),
        in_specs=[pl.BlockSpec((bm, D), lambda i: (i, 0)),
                  pl.BlockSpec((D, HID), lambda i: (0, 0)),
                  pl.BlockSpec((1, HID), lambda i: (0, 0))],
        out_specs=pl.BlockSpec((bm, HID), lambda i: (i, 0)),
        out_shape=jax.ShapeDtypeStruct((n, HID), jnp.float32),
    )(x, W, b.reshape(1, HID))


def _combine_body(relu, x_ref, s_ref, cnt_ref, basis_ref, comp_ref, root_ref,
                  b_ref, o_ref):
    acc = jnp.dot(x_ref[...], root_ref[...], preferred_element_type=jnp.float32)
    acc = acc + b_ref[...]
    comp = comp_ref[...]
    fn = []
    for r in range(R):
        cnt = jnp.maximum(cnt_ref[:, r:r + 1], 1.0)
        fn.append(s_ref[r] / cnt)
    for b in range(NB):
        gb = fn[0] * comp[0, b]
        for r in range(1, R):
            gb = gb + fn[r] * comp[r, b]
        acc = acc + jnp.dot(gb, basis_ref[b],
                            preferred_element_type=jnp.float32)
    if relu:
        acc = jnp.maximum(acc, 0.0)
    o_ref[...] = acc


def _combine(x, s, cnt, basis, comp, root, bias, relu):
    bm = 1000
    body = functools.partial(_combine_body, relu)
    return pl.pallas_call(
        body,
        grid=(N // bm,),
        in_specs=[pl.BlockSpec((bm, D), lambda i: (i, 0)),
                  pl.BlockSpec((R, bm, D), lambda i: (0, i, 0)),
                  pl.BlockSpec((bm, R), lambda i: (i, 0)),
                  pl.BlockSpec((NB, HID, HID), lambda i: (0, 0, 0)),
                  pl.BlockSpec((R, NB), lambda i: (0, 0)),
                  pl.BlockSpec((HID, HID), lambda i: (0, 0)),
                  pl.BlockSpec((1, HID), lambda i: (0, 0))],
        out_specs=pl.BlockSpec((bm, HID), lambda i: (i, 0)),
        out_shape=jax.ShapeDtypeStruct((N, HID), jnp.float32),
    )(x, s, cnt, basis, comp, root, bias.reshape(1, HID))


def _assemble_s(s_raw):
    v = s_raw.reshape(NSC, PLANE, D)
    return (v[0, :DUMP] + v[1, :DUMP]).reshape(R, NPAD, D)[:, :N, :]


def _assemble_cnt(cnt_raw):
    v = cnt_raw.reshape(NSC, NTILE, R, CB).sum(axis=0)
    v = v[:, :, :OWN].transpose(1, 0, 2).reshape(R, NPAD)[:, :N]
    return v.T


def kernel(x_user, x_item, edge_index, edge_type, Wp_user, bp_user, Wp_item,
           bp_item, basis0, comp0, root0, bias0, basis1, comp1, root1, bias1):
    src = edge_index[0]
    dst = edge_index[1]
    typ = edge_type
    zeros_z = jnp.zeros((ZR, D), jnp.float32)

    x0u = _project(x_user, Wp_user, bp_user)
    x0i = _project(x_item, Wp_item, bp_item)
    x0 = jnp.concatenate([x0u, x0i], axis=0)

    s_raw0, cnt_raw = _sc_segment_sums(x0, src, dst, typ, zeros_z)
    cnt = _assemble_cnt(cnt_raw)
    x1 = _combine(x0, _assemble_s(s_raw0), cnt, basis0, comp0, root0, bias0,
                  relu=True)

    s_raw1, _ = _sc_segment_sums(x1, src, dst, typ, zeros_z)
    x_out = _combine(x1, _assemble_s(s_raw1), cnt, basis1, comp1, root1,
                     bias1, relu=False)
    return (x_out, x0, x1)


# speedup vs baseline: 25.0205x; 25.0205x over previous
import functools

import jax
import jax.numpy as jnp
from jax import lax
from jax.experimental import pallas as pl
from jax.experimental.pallas import tpu as pltpu
from jax.experimental.pallas import tpu_sc as plsc

N_USER = 5000
N_ITEM = 5000
N = N_USER + N_ITEM
E = 160000
D = 256
R = 4
NB = 2
HID = 256
NTILE = 16
NSC = 2
NPAD = 10112
OWN = NPAD // NTILE
DUMP = R * NPAD
PLANE = DUMP + NTILE
EHALF = E // NSC
EB = 2000
BATCH = 64
CAP = 16384
RMAX = 2528 + BATCH + 16
CB = OWN
ZR = 64


def _sc_body(x_hbm, src_hbm, dst_hbm, typ_hbm, zeros_hbm, s_out, cnt_out,
             es, ed, et, gbuf, rbuf, kbuf, g2buf, r2buf, rows_v, sacc, zbuf,
             cntv):
    c = lax.axis_index("c")
    s = lax.axis_index("s")
    lo = s * OWN
    plane = c * PLANE

    pltpu.sync_copy(zeros_hbm, zbuf)
    for t in range(R):
        base = plane + t * NPAD + lo
        off = 0
        for nz in (ZR, ZR, ZR, ZR, ZR, ZR, ZR, ZR, ZR, OWN - 9 * ZR):
            pltpu.sync_copy(zbuf.at[pl.ds(0, nz)],
                            s_out.at[pl.ds(base + off, nz)])
            off += nz

    def zc(i, carry):
        cntv[pl.ds(i * 16, 16)] = jnp.zeros((16,), jnp.float32)
        return carry

    lax.fori_loop(0, (R * CB) // 16, zc, jnp.int32(0))

    ebase = c * EHALF
    zero16 = jnp.zeros((16,), jnp.int32)
    neg16 = jnp.full((16,), -1, jnp.int32)
    dump16 = jnp.full((16,), DUMP, jnp.int32) + plane + s
    iota16 = lax.iota(jnp.int32, 16)

    def flush(m):
        kbuf[pl.ds(m, 16)] = neg16
        nv = (m + 15) // 16

        def round_body(state):
            k, rem = state

            def rfilt(i, m2):
                kb16 = kbuf[pl.ds(i * 16, 16)]
                msk2 = kb16 == k
                g16 = gbuf[pl.ds(i * 16, 16)]
                r16 = rbuf[pl.ds(i * 16, 16)]
                inc2 = msk2.astype(jnp.int32)
                cs2 = plsc.cumsum(inc2)
                pos2 = m2 + cs2 - inc2
                plsc.store_scatter(g2buf, [pos2], g16, mask=msk2)
                plsc.store_scatter(r2buf, [pos2 // BATCH, pos2 % BATCH],
                                   r16, mask=msk2)
                return m2 + jnp.sum(inc2)

            m2 = lax.fori_loop(0, nv, rfilt, jnp.int32(0))

            for kk in range(BATCH // 16):
                g2buf[pl.ds(m2 + kk * 16, 16)] = zero16
                p2 = m2 + kk * 16 + iota16
                plsc.store_scatter(r2buf, [p2 // BATCH, p2 % BATCH], dump16)

            def gs(b, carry2):
                off = b * BATCH
                pltpu.sync_copy(s_out.at[r2buf.at[b]], sacc)
                pltpu.sync_copy(x_hbm.at[g2buf.at[pl.ds(off, BATCH)]], rows_v)

                def addrow(i, c3):
                    for kk in range(D // 16):
                        sl = pl.ds(kk * 16, 16)
                        sacc[i, sl] = sacc[i, sl] + rows_v[i, sl]
                    return c3

                lax.fori_loop(0, BATCH, addrow, jnp.int32(0))
                pltpu.sync_copy(sacc, s_out.at[r2buf.at[b]])
                return carry2

            nb = (m2 + BATCH - 1) // BATCH
            lax.fori_loop(0, nb, gs, jnp.int32(0))
            return (k + 1, rem - m2)

        lax.while_loop(lambda st: st[1] > 0, round_body, (jnp.int32(0), m))
        return jnp.int32(0)

    def edge_batch(eb, m_in):
        eoff = ebase + eb * EB
        pltpu.sync_copy(src_hbm.at[pl.ds(eoff, EB)], es)
        pltpu.sync_copy(dst_hbm.at[pl.ds(eoff, EB)], ed)
        pltpu.sync_copy(typ_hbm.at[pl.ds(eoff, EB)], et)

        def filt(i, m):
            d16 = ed[pl.ds(i * 16, 16)]
            msk = (d16 >= lo) & (d16 < lo + OWN)
            s16 = es[pl.ds(i * 16, 16)]
            t16 = et[pl.ds(i * 16, 16)]
            rowid = plane + t16 * NPAD + d16
            inc = msk.astype(jnp.int32)
            cs = plsc.cumsum(inc)
            pos = m + cs - inc
            plsc.store_scatter(gbuf, [pos], s16, mask=msk)
            plsc.store_scatter(rbuf, [pos], rowid, mask=msk)
            lidx = t16 * CB + (d16 - lo)
            rc, lastm = plsc.scan_count(lidx, mask=msk)
            okm = lastm & msk
            curall = plsc.load_gather(cntv, [lidx], mask=msk)
            grank = curall.astype(jnp.int32) + rc - 1
            plsc.store_scatter(kbuf, [pos], grank, mask=msk)
            cur = plsc.load_gather(cntv, [lidx], mask=okm)
            plsc.store_scatter(cntv, [lidx], cur + rc.astype(jnp.float32),
                               mask=okm)
            return m + jnp.sum(inc)

        m = lax.fori_loop(0, EB // 16, filt, m_in)
        return lax.cond(m > CAP - EB, flush, lambda mm: mm, m)

    m_fin = lax.fori_loop(0, EHALF // EB, edge_batch, jnp.int32(0))
    flush(m_fin)

    pltpu.sync_copy(cntv, cnt_out.at[c * NTILE + s])


def _sc_segment_sums(x, src, dst, typ, zeros_z):
    f = pl.kernel(
        _sc_body,
        out_type=(
            jax.ShapeDtypeStruct((NSC * PLANE, D), jnp.float32),
            jax.ShapeDtypeStruct((NSC * NTILE, R * CB), jnp.float32),
        ),
        mesh=plsc.VectorSubcoreMesh(core_axis_name="c", subcore_axis_name="s"),
        scratch_types=[
            pltpu.VMEM((EB,), jnp.int32),
            pltpu.VMEM((EB,), jnp.int32),
            pltpu.VMEM((EB,), jnp.int32),
            pltpu.VMEM((CAP + 16,), jnp.int32),
            pltpu.VMEM((CAP + 16,), jnp.int32),
            pltpu.VMEM((CAP + 16,), jnp.int32),
            pltpu.VMEM((RMAX,), jnp.int32),
            pltpu.VMEM((RMAX // BATCH + 1, BATCH), jnp.int32),
            pltpu.VMEM((BATCH, D), jnp.float32),
            pltpu.VMEM((BATCH, D), jnp.float32),
            pltpu.VMEM((ZR, D), jnp.float32),
            pltpu.VMEM((R * CB,), jnp.float32),
        ],
        compiler_params=pltpu.CompilerParams(needs_layout_passes=False),
    )
    return f(x, src, dst, typ, zeros_z)


def _proj_body(x_ref, w_ref, b_ref, o_ref):
    h = jnp.dot(x_ref[...], w_ref[...], preferred_element_type=jnp.float32)
    o_ref[...] = jnp.maximum(h + b_ref[...], 0.0)


def _project(x, W, b):
    n, bm = x.shape[0], 1000
    return pl.pallas_call(
        _proj_body,
        grid=(n // bm,),
        in_specs=[pl.BlockSpec((bm, D), lambda i: (i, 0)),
                  pl.BlockSpec((D, HID), lambda i: (0, 0)),
                  pl.BlockSpec((1, HID), lambda i: (0, 0))],
        out_specs=pl.BlockSpec((bm, HID), lambda i: (i, 0)),
        out_shape=jax.ShapeDtypeStruct((n, HID), jnp.float32),
    )(x, W, b.reshape(1, HID))


def _combine_body(relu, x_ref, s_ref, cnt_ref, basis_ref, comp_ref, root_ref,
                  b_ref, o_ref):
    acc = jnp.dot(x_ref[...], root_ref[...], preferred_element_type=jnp.float32)
    acc = acc + b_ref[...]
    comp = comp_ref[...]
    fn = []
    for r in range(R):
        cnt = jnp.maximum(cnt_ref[:, r:r + 1], 1.0)
        fn.append(s_ref[r] / cnt)
    for b in range(NB):
        gb = fn[0] * comp[0, b]
        for r in range(1, R):
            gb = gb + fn[r] * comp[r, b]
        acc = acc + jnp.dot(gb, basis_ref[b],
                            preferred_element_type=jnp.float32)
    if relu:
        acc = jnp.maximum(acc, 0.0)
    o_ref[...] = acc


def _combine(x, s, cnt, basis, comp, root, bias, relu):
    bm = 1000
    body = functools.partial(_combine_body, relu)
    return pl.pallas_call(
        body,
        grid=(N // bm,),
        in_specs=[pl.BlockSpec((bm, D), lambda i: (i, 0)),
                  pl.BlockSpec((R, bm, D), lambda i: (0, i, 0)),
                  pl.BlockSpec((bm, R), lambda i: (i, 0)),
                  pl.BlockSpec((NB, HID, HID), lambda i: (0, 0, 0)),
                  pl.BlockSpec((R, NB), lambda i: (0, 0)),
                  pl.BlockSpec((HID, HID), lambda i: (0, 0)),
                  pl.BlockSpec((1, HID), lambda i: (0, 0))],
        out_specs=pl.BlockSpec((bm, HID), lambda i: (i, 0)),
        out_shape=jax.ShapeDtypeStruct((N, HID), jnp.float32),
    )(x, s, cnt, basis, comp, root, bias.reshape(1, HID))


def _assemble_s(s_raw):
    v = s_raw.reshape(NSC, PLANE, D)
    return (v[0, :DUMP] + v[1, :DUMP]).reshape(R, NPAD, D)[:, :N, :]


def _assemble_cnt(cnt_raw):
    v = cnt_raw.reshape(NSC, NTILE, R, CB).sum(axis=0)
    v = v[:, :, :OWN].transpose(1, 0, 2).reshape(R, NPAD)[:, :N]
    return v.T


def kernel(x_user, x_item, edge_index, edge_type, Wp_user, bp_user, Wp_item,
           bp_item, basis0, comp0, root0, bias0, basis1, comp1, root1, bias1):
    src = edge_index[0]
    dst = edge_index[1]
    typ = edge_type
    zeros_z = jnp.zeros((ZR, D), jnp.float32)

    x0u = _project(x_user, Wp_user, bp_user)
    x0i = _project(x_item, Wp_item, bp_item)
    x0 = jnp.concatenate([x0u, x0i], axis=0)

    s_raw0, cnt_raw = _sc_segment_sums(x0, src, dst, typ, zeros_z)
    cnt = _assemble_cnt(cnt_raw)
    x1 = _combine(x0, _assemble_s(s_raw0), cnt, basis0, comp0, root0, bias0,
                  relu=True)

    s_raw1, _ = _sc_segment_sums(x1, src, dst, typ, zeros_z)
    x_out = _combine(x1, _assemble_s(s_raw1), cnt, basis1, comp1, root1,
                     bias1, relu=False)
    return (x_out, x0, x1)


# speedup vs baseline: 26.3003x; 1.0511x over previous
import functools

import jax
import jax.numpy as jnp
from jax import lax
from jax.experimental import pallas as pl
from jax.experimental.pallas import tpu as pltpu
from jax.experimental.pallas import tpu_sc as plsc

N_USER = 5000
N_ITEM = 5000
N = N_USER + N_ITEM
E = 160000
D = 256
R = 4
NB = 2
HID = 256
NTILE = 16
NSC = 2
NPAD = 10112
OWN = NPAD // NTILE
DUMP = R * NPAD
PLANE = DUMP + NTILE
EHALF = E // NSC
EB = 2000
BATCH = 64
CAP = 16384
RMAX = 2528 + BATCH + 16
CB = OWN
ZR = 64


def _sc_body(x_hbm, src_hbm, dst_hbm, typ_hbm, zeros_hbm, s_out, cnt_out,
             es, ed, et, gbuf, rbuf, kbuf, g2buf, r2buf, rows_v, sacc, zbuf,
             cntv, sem1, sem2):
    c = lax.axis_index("c")
    s = lax.axis_index("s")
    lo = s * OWN
    plane = c * PLANE

    pltpu.sync_copy(zeros_hbm, zbuf)
    for t in range(R):
        base = plane + t * NPAD + lo
        off = 0
        for nz in (ZR, ZR, ZR, ZR, ZR, ZR, ZR, ZR, ZR, OWN - 9 * ZR):
            pltpu.sync_copy(zbuf.at[pl.ds(0, nz)],
                            s_out.at[pl.ds(base + off, nz)])
            off += nz

    def zc(i, carry):
        cntv[pl.ds(i * 16, 16)] = jnp.zeros((16,), jnp.float32)
        return carry

    lax.fori_loop(0, (R * CB) // 16, zc, jnp.int32(0))

    ebase = c * EHALF
    zero16 = jnp.zeros((16,), jnp.int32)
    neg16 = jnp.full((16,), -1, jnp.int32)
    dump16 = jnp.full((16,), DUMP, jnp.int32) + plane + s
    iota16 = lax.iota(jnp.int32, 16)

    def flush(m):
        kbuf[pl.ds(m, 16)] = neg16
        nv = (m + 15) // 16

        def round_body(state):
            k, rem = state

            def rfilt(i, m2):
                kb16 = kbuf[pl.ds(i * 16, 16)]
                msk2 = kb16 == k
                g16 = gbuf[pl.ds(i * 16, 16)]
                r16 = rbuf[pl.ds(i * 16, 16)]
                inc2 = msk2.astype(jnp.int32)
                cs2 = plsc.cumsum(inc2)
                pos2 = m2 + cs2 - inc2
                plsc.store_scatter(g2buf, [pos2], g16, mask=msk2)
                plsc.store_scatter(r2buf, [pos2 // BATCH, pos2 % BATCH],
                                   r16, mask=msk2)
                return m2 + jnp.sum(inc2)

            m2 = lax.fori_loop(0, nv, rfilt, jnp.int32(0))

            for kk in range(BATCH // 16):
                g2buf[pl.ds(m2 + kk * 16, 16)] = zero16
                p2 = m2 + kk * 16 + iota16
                plsc.store_scatter(r2buf, [p2 // BATCH, p2 % BATCH], dump16)

            def gs(b, carry2):
                off = b * BATCH
                d1 = pltpu.async_copy(s_out.at[r2buf.at[b]], sacc, sem1)
                d2 = pltpu.async_copy(x_hbm.at[g2buf.at[pl.ds(off, BATCH)]],
                                      rows_v, sem2)
                d1.wait()
                d2.wait()

                def addrow(i, c3):
                    for kk in range(D // 16):
                        sl = pl.ds(kk * 16, 16)
                        sacc[i, sl] = sacc[i, sl] + rows_v[i, sl]
                    return c3

                lax.fori_loop(0, BATCH, addrow, jnp.int32(0))
                pltpu.sync_copy(sacc, s_out.at[r2buf.at[b]])
                return carry2

            nb = (m2 + BATCH - 1) // BATCH
            lax.fori_loop(0, nb, gs, jnp.int32(0))
            return (k + 1, rem - m2)

        lax.while_loop(lambda st: st[1] > 0, round_body, (jnp.int32(0), m))
        return jnp.int32(0)

    def edge_batch(eb, m_in):
        eoff = ebase + eb * EB
        pltpu.sync_copy(src_hbm.at[pl.ds(eoff, EB)], es)
        pltpu.sync_copy(dst_hbm.at[pl.ds(eoff, EB)], ed)
        pltpu.sync_copy(typ_hbm.at[pl.ds(eoff, EB)], et)

        def filt(i, m):
            d16 = ed[pl.ds(i * 16, 16)]
            msk = (d16 >= lo) & (d16 < lo + OWN)
            s16 = es[pl.ds(i * 16, 16)]
            t16 = et[pl.ds(i * 16, 16)]
            rowid = plane + t16 * NPAD + d16
            inc = msk.astype(jnp.int32)
            cs = plsc.cumsum(inc)
            pos = m + cs - inc
            plsc.store_scatter(gbuf, [pos], s16, mask=msk)
            plsc.store_scatter(rbuf, [pos], rowid, mask=msk)
            lidx = t16 * CB + (d16 - lo)
            rc, lastm = plsc.scan_count(lidx, mask=msk)
            okm = lastm & msk
            curall = plsc.load_gather(cntv, [lidx], mask=msk)
            grank = curall.astype(jnp.int32) + rc - 1
            plsc.store_scatter(kbuf, [pos], grank, mask=msk)
            cur = plsc.load_gather(cntv, [lidx], mask=okm)
            plsc.store_scatter(cntv, [lidx], cur + rc.astype(jnp.float32),
                               mask=okm)
            return m + jnp.sum(inc)

        m = lax.fori_loop(0, EB // 16, filt, m_in)
        return lax.cond(m > CAP - EB, flush, lambda mm: mm, m)

    m_fin = lax.fori_loop(0, EHALF // EB, edge_batch, jnp.int32(0))
    flush(m_fin)

    pltpu.sync_copy(cntv, cnt_out.at[c * NTILE + s])


def _sc_segment_sums(x, src, dst, typ, zeros_z):
    f = pl.kernel(
        _sc_body,
        out_type=(
            jax.ShapeDtypeStruct((NSC * PLANE, D), jnp.float32),
            jax.ShapeDtypeStruct((NSC * NTILE, R * CB), jnp.float32),
        ),
        mesh=plsc.VectorSubcoreMesh(core_axis_name="c", subcore_axis_name="s"),
        scratch_types=[
            pltpu.VMEM((EB,), jnp.int32),
            pltpu.VMEM((EB,), jnp.int32),
            pltpu.VMEM((EB,), jnp.int32),
            pltpu.VMEM((CAP + 16,), jnp.int32),
            pltpu.VMEM((CAP + 16,), jnp.int32),
            pltpu.VMEM((CAP + 16,), jnp.int32),
            pltpu.VMEM((RMAX,), jnp.int32),
            pltpu.VMEM((RMAX // BATCH + 1, BATCH), jnp.int32),
            pltpu.VMEM((BATCH, D), jnp.float32),
            pltpu.VMEM((BATCH, D), jnp.float32),
            pltpu.VMEM((ZR, D), jnp.float32),
            pltpu.VMEM((R * CB,), jnp.float32),
            pltpu.SemaphoreType.DMA,
            pltpu.SemaphoreType.DMA,
        ],
        compiler_params=pltpu.CompilerParams(needs_layout_passes=False),
    )
    return f(x, src, dst, typ, zeros_z)


def _proj_body(x_ref, w_ref, b_ref, o_ref):
    h = jnp.dot(x_ref[...], w_ref[...], preferred_element_type=jnp.float32)
    o_ref[...] = jnp.maximum(h + b_ref[...], 0.0)


def _project(x, W, b):
    n, bm = x.shape[0], 1000
    return pl.pallas_call(
        _proj_body,
        grid=(n // bm,),
        in_specs=[pl.BlockSpec((bm, D), lambda i: (i, 0)),
                  pl.BlockSpec((D, HID), lambda i: (0, 0)),
                  pl.BlockSpec((1, HID), lambda i: (0, 0))],
        out_specs=pl.BlockSpec((bm, HID), lambda i: (i, 0)),
        out_shape=jax.ShapeDtypeStruct((n, HID), jnp.float32),
    )(x, W, b.reshape(1, HID))


def _combine_body(relu, x_ref, s_ref, cnt_ref, basis_ref, comp_ref, root_ref,
                  b_ref, o_ref):
    acc = jnp.dot(x_ref[...], root_ref[...], preferred_element_type=jnp.float32)
    acc = acc + b_ref[...]
    comp = comp_ref[...]
    fn = []
    for r in range(R):
        cnt = jnp.maximum(cnt_ref[:, r:r + 1], 1.0)
        fn.append(s_ref[r] / cnt)
    for b in range(NB):
        gb = fn[0] * comp[0, b]
        for r in range(1, R):
            gb = gb + fn[r] * comp[r, b]
        acc = acc + jnp.dot(gb, basis_ref[b],
                            preferred_element_type=jnp.float32)
    if relu:
        acc = jnp.maximum(acc, 0.0)
    o_ref[...] = acc


def _combine(x, s, cnt, basis, comp, root, bias, relu):
    bm = 1000
    body = functools.partial(_combine_body, relu)
    return pl.pallas_call(
        body,
        grid=(N // bm,),
        in_specs=[pl.BlockSpec((bm, D), lambda i: (i, 0)),
                  pl.BlockSpec((R, bm, D), lambda i: (0, i, 0)),
                  pl.BlockSpec((bm, R), lambda i: (i, 0)),
                  pl.BlockSpec((NB, HID, HID), lambda i: (0, 0, 0)),
                  pl.BlockSpec((R, NB), lambda i: (0, 0)),
                  pl.BlockSpec((HID, HID), lambda i: (0, 0)),
                  pl.BlockSpec((1, HID), lambda i: (0, 0))],
        out_specs=pl.BlockSpec((bm, HID), lambda i: (i, 0)),
        out_shape=jax.ShapeDtypeStruct((N, HID), jnp.float32),
    )(x, s, cnt, basis, comp, root, bias.reshape(1, HID))


def _assemble_s(s_raw):
    v = s_raw.reshape(NSC, PLANE, D)
    return (v[0, :DUMP] + v[1, :DUMP]).reshape(R, NPAD, D)[:, :N, :]


def _assemble_cnt(cnt_raw):
    v = cnt_raw.reshape(NSC, NTILE, R, CB).sum(axis=0)
    v = v[:, :, :OWN].transpose(1, 0, 2).reshape(R, NPAD)[:, :N]
    return v.T


def kernel(x_user, x_item, edge_index, edge_type, Wp_user, bp_user, Wp_item,
           bp_item, basis0, comp0, root0, bias0, basis1, comp1, root1, bias1):
    src = edge_index[0]
    dst = edge_index[1]
    typ = edge_type
    zeros_z = jnp.zeros((ZR, D), jnp.float32)

    x0u = _project(x_user, Wp_user, bp_user)
    x0i = _project(x_item, Wp_item, bp_item)
    x0 = jnp.concatenate([x0u, x0i], axis=0)

    s_raw0, cnt_raw = _sc_segment_sums(x0, src, dst, typ, zeros_z)
    cnt = _assemble_cnt(cnt_raw)
    x1 = _combine(x0, _assemble_s(s_raw0), cnt, basis0, comp0, root0, bias0,
                  relu=True)

    s_raw1, _ = _sc_segment_sums(x1, src, dst, typ, zeros_z)
    x_out = _combine(x1, _assemble_s(s_raw1), cnt, basis1, comp1, root1,
                     bias1, relu=False)
    return (x_out, x0, x1)


# speedup vs baseline: 27.5872x; 1.0489x over previous
import functools

import jax
import jax.numpy as jnp
from jax import lax
from jax.experimental import pallas as pl
from jax.experimental.pallas import tpu as pltpu
from jax.experimental.pallas import tpu_sc as plsc

N_USER = 5000
N_ITEM = 5000
N = N_USER + N_ITEM
E = 160000
D = 256
R = 4
NB = 2
HID = 256
NTILE = 16
NSC = 2
NPAD = 10112
OWN = NPAD // NTILE
DUMP = R * NPAD
PLANE = DUMP + NTILE
EHALF = E // NSC
EB = 2000
BATCH = 64
CAP = 16384
RMAX = 2528 + BATCH + 16
CB = OWN
ZR = 64


def _sc_body(x_hbm, src_hbm, dst_hbm, typ_hbm, zeros_hbm, s_out, cnt_out,
             es, ed, et, gbuf, rbuf, kbuf, g2buf, r2buf, rows_v, sacc, zbuf,
             cntv, sem1, sem2):
    c = lax.axis_index("c")
    s = lax.axis_index("s")
    lo = s * OWN
    plane = c * PLANE

    pltpu.sync_copy(zeros_hbm, zbuf)
    for t in range(R):
        base = plane + t * NPAD + lo
        off = 0
        for nz in (ZR, ZR, ZR, ZR, ZR, ZR, ZR, ZR, ZR, OWN - 9 * ZR):
            pltpu.sync_copy(zbuf.at[pl.ds(0, nz)],
                            s_out.at[pl.ds(base + off, nz)])
            off += nz

    def zc(i, carry):
        cntv[pl.ds(i * 16, 16)] = jnp.zeros((16,), jnp.float32)
        return carry

    lax.fori_loop(0, (R * CB) // 16, zc, jnp.int32(0))

    ebase = c * EHALF
    zero16 = jnp.zeros((16,), jnp.int32)
    neg16 = jnp.full((16,), -1, jnp.int32)
    dump16 = jnp.full((16,), DUMP, jnp.int32) + plane + s
    iota16 = lax.iota(jnp.int32, 16)

    def flush(m):
        kbuf[pl.ds(m, 16)] = neg16
        nv = (m + 15) // 16

        def round_body(state):
            k, rem = state

            def rfilt(i, m2):
                kb16 = kbuf[pl.ds(i * 16, 16)]
                msk2 = kb16 == k
                g16 = gbuf[pl.ds(i * 16, 16)]
                r16 = rbuf[pl.ds(i * 16, 16)]
                inc2 = msk2.astype(jnp.int32)
                cs2 = plsc.cumsum(inc2)
                pos2 = m2 + cs2 - inc2
                plsc.store_scatter(g2buf, [pos2], g16, mask=msk2)
                plsc.store_scatter(r2buf, [pos2 // BATCH, pos2 % BATCH],
                                   r16, mask=msk2)
                return m2 + jnp.sum(inc2)

            m2 = lax.fori_loop(0, nv, rfilt, jnp.int32(0))

            for kk in range(BATCH // 16):
                g2buf[pl.ds(m2 + kk * 16, 16)] = zero16
                p2 = m2 + kk * 16 + iota16
                plsc.store_scatter(r2buf, [p2 // BATCH, p2 % BATCH], dump16)

            def gs0(b, carry2):
                off = b * BATCH
                pltpu.sync_copy(x_hbm.at[g2buf.at[pl.ds(off, BATCH)]],
                                rows_v)
                pltpu.sync_copy(rows_v, s_out.at[r2buf.at[b]])
                return carry2

            def gs(b, carry2):
                off = b * BATCH
                d1 = pltpu.async_copy(s_out.at[r2buf.at[b]], sacc, sem1)
                d2 = pltpu.async_copy(x_hbm.at[g2buf.at[pl.ds(off, BATCH)]],
                                      rows_v, sem2)
                d1.wait()
                d2.wait()

                def addrow(i, c3):
                    for kk in range(D // 16):
                        sl = pl.ds(kk * 16, 16)
                        sacc[i, sl] = sacc[i, sl] + rows_v[i, sl]
                    return c3

                lax.fori_loop(0, BATCH, addrow, jnp.int32(0))
                pltpu.sync_copy(sacc, s_out.at[r2buf.at[b]])
                return carry2

            nb = (m2 + BATCH - 1) // BATCH

            def all0(c4):
                lax.fori_loop(0, nb, gs0, jnp.int32(0))
                return c4

            def allk(c4):
                lax.fori_loop(0, nb, gs, jnp.int32(0))
                return c4

            lax.cond(k == 0, all0, allk, jnp.int32(0))
            return (k + 1, rem - m2)

        lax.while_loop(lambda st: st[1] > 0, round_body, (jnp.int32(0), m))
        return jnp.int32(0)

    def edge_batch(eb, m_in):
        eoff = ebase + eb * EB
        pltpu.sync_copy(src_hbm.at[pl.ds(eoff, EB)], es)
        pltpu.sync_copy(dst_hbm.at[pl.ds(eoff, EB)], ed)
        pltpu.sync_copy(typ_hbm.at[pl.ds(eoff, EB)], et)

        def filt(i, m):
            d16 = ed[pl.ds(i * 16, 16)]
            msk = (d16 >= lo) & (d16 < lo + OWN)
            s16 = es[pl.ds(i * 16, 16)]
            t16 = et[pl.ds(i * 16, 16)]
            rowid = plane + t16 * NPAD + d16
            inc = msk.astype(jnp.int32)
            cs = plsc.cumsum(inc)
            pos = m + cs - inc
            plsc.store_scatter(gbuf, [pos], s16, mask=msk)
            plsc.store_scatter(rbuf, [pos], rowid, mask=msk)
            lidx = t16 * CB + (d16 - lo)
            rc, lastm = plsc.scan_count(lidx, mask=msk)
            okm = lastm & msk
            curall = plsc.load_gather(cntv, [lidx], mask=msk)
            grank = curall.astype(jnp.int32) + rc - 1
            plsc.store_scatter(kbuf, [pos], grank, mask=msk)
            cur = plsc.load_gather(cntv, [lidx], mask=okm)
            plsc.store_scatter(cntv, [lidx], cur + rc.astype(jnp.float32),
                               mask=okm)
            return m + jnp.sum(inc)

        m = lax.fori_loop(0, EB // 16, filt, m_in)
        return lax.cond(m > CAP - EB, flush, lambda mm: mm, m)

    m_fin = lax.fori_loop(0, EHALF // EB, edge_batch, jnp.int32(0))
    flush(m_fin)

    pltpu.sync_copy(cntv, cnt_out.at[c * NTILE + s])


def _sc_segment_sums(x, src, dst, typ, zeros_z):
    f = pl.kernel(
        _sc_body,
        out_type=(
            jax.ShapeDtypeStruct((NSC * PLANE, D), jnp.float32),
            jax.ShapeDtypeStruct((NSC * NTILE, R * CB), jnp.float32),
        ),
        mesh=plsc.VectorSubcoreMesh(core_axis_name="c", subcore_axis_name="s"),
        scratch_types=[
            pltpu.VMEM((EB,), jnp.int32),
            pltpu.VMEM((EB,), jnp.int32),
            pltpu.VMEM((EB,), jnp.int32),
            pltpu.VMEM((CAP + 16,), jnp.int32),
            pltpu.VMEM((CAP + 16,), jnp.int32),
            pltpu.VMEM((CAP + 16,), jnp.int32),
            pltpu.VMEM((RMAX,), jnp.int32),
            pltpu.VMEM((RMAX // BATCH + 1, BATCH), jnp.int32),
            pltpu.VMEM((BATCH, D), jnp.float32),
            pltpu.VMEM((BATCH, D), jnp.float32),
            pltpu.VMEM((ZR, D), jnp.float32),
            pltpu.VMEM((R * CB,), jnp.float32),
            pltpu.SemaphoreType.DMA,
            pltpu.SemaphoreType.DMA,
        ],
        compiler_params=pltpu.CompilerParams(needs_layout_passes=False),
    )
    return f(x, src, dst, typ, zeros_z)


def _proj_body(x_ref, w_ref, b_ref, o_ref):
    h = jnp.dot(x_ref[...], w_ref[...], preferred_element_type=jnp.float32)
    o_ref[...] = jnp.maximum(h + b_ref[...], 0.0)


def _project(x, W, b):
    n, bm = x.shape[0], 1000
    return pl.pallas_call(
        _proj_body,
        grid=(n // bm,),
        in_specs=[pl.BlockSpec((bm, D), lambda i: (i, 0)),
                  pl.BlockSpec((D, HID), lambda i: (0, 0)),
                  pl.BlockSpec((1, HID), lambda i: (0, 0))],
        out_specs=pl.BlockSpec((bm, HID), lambda i: (i, 0)),
        out_shape=jax.ShapeDtypeStruct((n, HID), jnp.float32),
    )(x, W, b.reshape(1, HID))


def _combine_body(relu, x_ref, s_ref, cnt_ref, basis_ref, comp_ref, root_ref,
                  b_ref, o_ref):
    acc = jnp.dot(x_ref[...], root_ref[...], preferred_element_type=jnp.float32)
    acc = acc + b_ref[...]
    comp = comp_ref[...]
    fn = []
    for r in range(R):
        cnt = jnp.maximum(cnt_ref[:, r:r + 1], 1.0)
        fn.append(s_ref[r] / cnt)
    for b in range(NB):
        gb = fn[0] * comp[0, b]
        for r in range(1, R):
            gb = gb + fn[r] * comp[r, b]
        acc = acc + jnp.dot(gb, basis_ref[b],
                            preferred_element_type=jnp.float32)
    if relu:
        acc = jnp.maximum(acc, 0.0)
    o_ref[...] = acc


def _combine(x, s, cnt, basis, comp, root, bias, relu):
    bm = 1000
    body = functools.partial(_combine_body, relu)
    return pl.pallas_call(
        body,
        grid=(N // bm,),
        in_specs=[pl.BlockSpec((bm, D), lambda i: (i, 0)),
                  pl.BlockSpec((R, bm, D), lambda i: (0, i, 0)),
                  pl.BlockSpec((bm, R), lambda i: (i, 0)),
                  pl.BlockSpec((NB, HID, HID), lambda i: (0, 0, 0)),
                  pl.BlockSpec((R, NB), lambda i: (0, 0)),
                  pl.BlockSpec((HID, HID), lambda i: (0, 0)),
                  pl.BlockSpec((1, HID), lambda i: (0, 0))],
        out_specs=pl.BlockSpec((bm, HID), lambda i: (i, 0)),
        out_shape=jax.ShapeDtypeStruct((N, HID), jnp.float32),
    )(x, s, cnt, basis, comp, root, bias.reshape(1, HID))


def _assemble_s(s_raw):
    v = s_raw.reshape(NSC, PLANE, D)
    return (v[0, :DUMP] + v[1, :DUMP]).reshape(R, NPAD, D)[:, :N, :]


def _assemble_cnt(cnt_raw):
    v = cnt_raw.reshape(NSC, NTILE, R, CB).sum(axis=0)
    v = v[:, :, :OWN].transpose(1, 0, 2).reshape(R, NPAD)[:, :N]
    return v.T


def kernel(x_user, x_item, edge_index, edge_type, Wp_user, bp_user, Wp_item,
           bp_item, basis0, comp0, root0, bias0, basis1, comp1, root1, bias1):
    src = edge_index[0]
    dst = edge_index[1]
    typ = edge_type
    zeros_z = jnp.zeros((ZR, D), jnp.float32)

    x0u = _project(x_user, Wp_user, bp_user)
    x0i = _project(x_item, Wp_item, bp_item)
    x0 = jnp.concatenate([x0u, x0i], axis=0)

    s_raw0, cnt_raw = _sc_segment_sums(x0, src, dst, typ, zeros_z)
    cnt = _assemble_cnt(cnt_raw)
    x1 = _combine(x0, _assemble_s(s_raw0), cnt, basis0, comp0, root0, bias0,
                  relu=True)

    s_raw1, _ = _sc_segment_sums(x1, src, dst, typ, zeros_z)
    x_out = _combine(x1, _assemble_s(s_raw1), cnt, basis1, comp1, root1,
                     bias1, relu=False)
    return (x_out, x0, x1)
